# baseline (speedup 1.0000x reference)
import jax
import jax.numpy as jnp
from jax import lax
from jax.experimental import pallas as pl
from jax.experimental.pallas import tpu as pltpu
from jax.experimental.pallas import tpu_sc as plsc

B = 16384
NUM_MOVIES = 1000000
MOVIE_DIM = 64
CAT_DIM = 32
L = 10
NC = 2
NS = 16
NW = NC * NS
BPW = B // NW
CH = 128
NCH = BPW // CH
CHC = 16
NQ = CH // CHC
LANES = 16
NG = BPW // LANES


def _movie_body(mid_hbm, movies_hbm, out_hbm, mid_v, grp_v, comb_v,
                sem_g0, sem_g1, sem_g2, sem_g3):
    cid = lax.axis_index("c")
    sid = lax.axis_index("s")
    wid = sid * NC + cid
    base = wid * BPW

    pltpu.sync_copy(mid_hbm.at[wid], mid_v)
    sems = (sem_g0, sem_g1, sem_g2, sem_g3)
    DEPTH = 4

    def issue_group(g, p):
        mv = mid_v[g // 8, pl.ds((g % 8) * LANES, LANES)]
        for j in range(LANES):
            idx = mv[j]
            g8 = pl.multiple_of((idx // 8) * 8, 8)
            pltpu.async_copy(movies_hbm.at[pl.ds(g8, 8)],
                             grp_v.at[p, j], sems[p])

    for p in range(DEPTH):
        issue_group(p, p)

    for c in range(NCH):
        def g4_body(k, carry):
            for p in range(DEPTH):
                g = c * 8 + k * DEPTH + p
                mv = mid_v[c, pl.ds((k * DEPTH + p) * LANES, LANES)]
                for j in range(LANES):
                    pltpu.make_async_copy(movies_hbm.at[pl.ds(0, 8)],
                                          grp_v.at[p, j], sems[p]).wait()
                    sub = mv[j] % 8
                    for h in range(MOVIE_DIM // LANES):
                        comb_v[(k * DEPTH + p) * LANES + j,
                               pl.ds(h * LANES, LANES)] = (
                            grp_v[p, j, sub, pl.ds(h * LANES, LANES)])

                @pl.when(g < NG - DEPTH)
                def _issue_next():
                    issue_group(g + DEPTH, p)

            return carry

        lax.fori_loop(0, 8 // DEPTH, g4_body, 0)
        pltpu.sync_copy(comb_v, out_hbm.at[pl.ds(base + c * CH, CH)])


_movie_call = pl.kernel(
    _movie_body,
    out_type=[jax.ShapeDtypeStruct((B, MOVIE_DIM), jnp.float32)],
    mesh=plsc.VectorSubcoreMesh(core_axis_name="c", subcore_axis_name="s"),
    compiler_params=pltpu.CompilerParams(use_tc_tiling_on_sc=True),
    scratch_types=[
        pltpu.VMEM((NCH, CH), jnp.int32),
        pltpu.VMEM((4, LANES, 8, MOVIE_DIM), jnp.float32),
        pltpu.VMEM((CH, MOVIE_DIM), jnp.float32),
        pltpu.SemaphoreType.DMA,
        pltpu.SemaphoreType.DMA,
        pltpu.SemaphoreType.DMA,
        pltpu.SemaphoreType.DMA,
    ],
)


def _cat_bias_body(mid_hbm, mct_hbm, cats_hbm, bias_hbm,
                   out_cat_hbm, out_bias_hbm,
                   mid_v, cats_v, g_v, outc_v, bias_c,
                   sem_b, sem_c):
    cid = lax.axis_index("c")
    sid = lax.axis_index("s")
    wid = sid * NC + cid
    base = wid * BPW

    pltpu.sync_copy(mid_hbm.at[wid], mid_v)
    pltpu.sync_copy(mct_hbm.at[wid], cats_v)

    for c in range(NCH):
        bias_dma = pltpu.async_copy(
            bias_hbm.at[mid_v.at[c]], bias_c, sem_b)

        def q_body(q, carry):
            cat_dmas = [
                pltpu.async_copy(
                    cats_hbm.at[cats_v.at[pl.ds((c * L + l) * CH + q * CHC,
                                                CHC)]],
                    g_v.at[l], sem_c)
                for l in range(L)
            ]
            for dma in cat_dmas:
                dma.wait()

            cnt = jnp.zeros((LANES,), jnp.float32)
            for l in range(L):
                cv = cats_v[pl.ds((c * L + l) * CH + q * CHC, LANES)]
                cnt = cnt + jnp.where(cv != 0, 1.0, 0.0)
            recip = 1.0 / jnp.maximum(cnt, 1.0)
            for j in range(LANES):
                rp = recip[j]
                for h in range(CAT_DIM // LANES):
                    s = g_v[0, j, pl.ds(h * LANES, LANES)]
                    for l in range(1, L):
                        s = s + g_v[l, j, pl.ds(h * LANES, LANES)]
                    outc_v[q * CHC + j,
                           pl.ds(h * LANES, LANES)] = s * rp
            return carry

        lax.fori_loop(0, NQ, q_body, 0)
        pltpu.sync_copy(outc_v, out_cat_hbm.at[pl.ds(base + c * CH, CH)])

        bias_dma.wait()
        pltpu.sync_copy(bias_c, out_bias_hbm.at[pl.ds(base + c * CH, CH)])


_cat_bias_call = pl.kernel(
    _cat_bias_body,
    out_type=[
        jax.ShapeDtypeStruct((B, CAT_DIM), jnp.float32),
        jax.ShapeDtypeStruct((B,), jnp.float32),
    ],
    mesh=plsc.VectorSubcoreMesh(core_axis_name="c", subcore_axis_name="s"),
    compiler_params=pltpu.CompilerParams(use_tc_tiling_on_sc=False),
    scratch_types=[
        pltpu.VMEM((NCH, CH), jnp.int32),
        pltpu.VMEM((NCH * L * CH,), jnp.int32),
        pltpu.VMEM((L, CHC, CAT_DIM), jnp.float32),
        pltpu.VMEM((CH, CAT_DIM), jnp.float32),
        pltpu.VMEM((CH,), jnp.float32),
        pltpu.SemaphoreType.DMA,
        pltpu.SemaphoreType.DMA,
    ],
)


def kernel(movie_id, movie_categories, emb_movies_W, emb_cats_W, bias_W):
    mid = movie_id.astype(jnp.int32).reshape(NW, NCH, CH)
    mct = (movie_categories.astype(jnp.int32).T
           .reshape(L, NW, NCH, CH).transpose(1, 2, 0, 3)
           .reshape(NW, NCH * L * CH))
    bias_flat = bias_W[:, 0]
    (mv,) = _movie_call(mid, emb_movies_W)
    cat, bias = _cat_bias_call(mid, mct, emb_cats_W, bias_flat)
    return jnp.concatenate([mv, cat], axis=1), bias


# speedup vs baseline: 1.0852x; 1.0852x over previous
import jax
import jax.numpy as jnp
from jax import lax
from jax.experimental import pallas as pl
from jax.experimental.pallas import tpu as pltpu
from jax.experimental.pallas import tpu_sc as plsc

B = 16384
NUM_MOVIES = 1000000
MOVIE_DIM = 64
CAT_DIM = 32
L = 10
NC = 2
NS = 16
NW = NC * NS
BPW = B // NW
CH = 128
NCH = BPW // CH
CHC = 16
NQ = CH // CHC
LANES = 16
NG = BPW // LANES


def _movie_body(mid_hbm, movies_hbm, bias_hbm, out_hbm, out_bias_hbm,
                mid_v, grp_v, bgrp_v, comb_v, bias_cv,
                sem_g0, sem_g1, sem_g2, sem_g3):
    cid = lax.axis_index("c")
    sid = lax.axis_index("s")
    wid = sid * NC + cid
    base = wid * BPW

    pltpu.sync_copy(mid_hbm.at[wid], mid_v)
    sems = (sem_g0, sem_g1, sem_g2, sem_g3)
    DEPTH = 4
    lane_iota = lax.iota(jnp.int32, LANES)

    def issue_group(g, p):
        mv = mid_v[g // 8, pl.ds((g % 8) * LANES, LANES)]
        for j in range(LANES):
            idx = mv[j]
            g8 = pl.multiple_of((idx // 8) * 8, 8)
            pltpu.async_copy(movies_hbm.at[pl.ds(g8, 8)],
                             grp_v.at[p, j], sems[p])
            b128 = pl.multiple_of((idx // 128) * 128, 128)
            pltpu.async_copy(bias_hbm.at[0, pl.ds(b128, 128)],
                             bgrp_v.at[p, j], sems[p])

    for p in range(DEPTH):
        issue_group(p, p)

    def g4_body(k, carry):
        for p in range(DEPTH):
            gk = k * DEPTH + p
            row0 = (gk % 8) * LANES
            mv = mid_v[gk // 8, pl.ds(row0, LANES)]
            vals = jnp.zeros((LANES,), jnp.float32)
            for j in range(LANES):
                pltpu.make_async_copy(movies_hbm.at[pl.ds(0, 8)],
                                      grp_v.at[p, j], sems[p]).wait()
                pltpu.make_async_copy(bias_hbm.at[0, pl.ds(0, 128)],
                                      bgrp_v.at[p, j], sems[p]).wait()
                idx = mv[j]
                sub = idx % 8
                for h in range(MOVIE_DIM // LANES):
                    comb_v[row0 + j, pl.ds(h * LANES, LANES)] = (
                        grp_v[p, j, sub, pl.ds(h * LANES, LANES)])
                sub16 = ((idx % 128) // 16) * 16
                bvec = bgrp_v[p, j, pl.ds(sub16, LANES)]
                picked = jnp.take_along_axis(
                    bvec, jnp.full((LANES,), idx % 16), axis=0)
                vals = jnp.where(lane_iota == j, picked, vals)

            bias_cv[pl.ds(row0, LANES)] = vals

            @pl.when(gk < NG - DEPTH)
            def _issue_next():
                issue_group(gk + DEPTH, p)

        @pl.when(k % 2 == 1)
        def _write_chunk():
            cdyn = k // 2
            pltpu.sync_copy(comb_v,
                            out_hbm.at[pl.ds(base + cdyn * CH, CH)])
            pltpu.sync_copy(bias_cv,
                            out_bias_hbm.at[pl.ds(base + cdyn * CH, CH)])

        return carry

    lax.fori_loop(0, NG // DEPTH, g4_body, 0)


_movie_call = pl.kernel(
    _movie_body,
    out_type=[
        jax.ShapeDtypeStruct((B, MOVIE_DIM), jnp.float32),
        jax.ShapeDtypeStruct((B,), jnp.float32),
    ],
    mesh=plsc.VectorSubcoreMesh(core_axis_name="c", subcore_axis_name="s"),
    compiler_params=pltpu.CompilerParams(use_tc_tiling_on_sc=True),
    scratch_types=[
        pltpu.VMEM((NCH, CH), jnp.int32),
        pltpu.VMEM((4, LANES, 8, MOVIE_DIM), jnp.float32),
        pltpu.VMEM((4, LANES, 128), jnp.float32),
        pltpu.VMEM((CH, MOVIE_DIM), jnp.float32),
        pltpu.VMEM((CH,), jnp.float32),
        pltpu.SemaphoreType.DMA,
        pltpu.SemaphoreType.DMA,
        pltpu.SemaphoreType.DMA,
        pltpu.SemaphoreType.DMA,
    ],
)


def _cat_body(mct_hbm, cats_hbm, out_cat_hbm,
              cats_v, g_v, outc_v, sem_c):
    cid = lax.axis_index("c")
    sid = lax.axis_index("s")
    wid = sid * NC + cid
    base = wid * BPW

    pltpu.sync_copy(mct_hbm.at[wid], cats_v)

    for c in range(NCH):
        def q_body(q, carry):
            cat_dmas = [
                pltpu.async_copy(
                    cats_hbm.at[cats_v.at[pl.ds((c * L + l) * CH + q * CHC,
                                                CHC)]],
                    g_v.at[l], sem_c)
                for l in range(L)
            ]
            for dma in cat_dmas:
                dma.wait()

            cnt = jnp.zeros((LANES,), jnp.float32)
            for l in range(L):
                cv = cats_v[pl.ds((c * L + l) * CH + q * CHC, LANES)]
                cnt = cnt + jnp.where(cv != 0, 1.0, 0.0)
            recip = 1.0 / jnp.maximum(cnt, 1.0)
            for j in range(LANES):
                rp = recip[j]
                for h in range(CAT_DIM // LANES):
                    s = g_v[0, j, pl.ds(h * LANES, LANES)]
                    for l in range(1, L):
                        s = s + g_v[l, j, pl.ds(h * LANES, LANES)]
                    outc_v[q * CHC + j,
                           pl.ds(h * LANES, LANES)] = s * rp
            return carry

        lax.fori_loop(0, NQ, q_body, 0)
        pltpu.sync_copy(outc_v, out_cat_hbm.at[pl.ds(base + c * CH, CH)])


_cat_call = pl.kernel(
    _cat_body,
    out_type=[jax.ShapeDtypeStruct((B, CAT_DIM), jnp.float32)],
    mesh=plsc.VectorSubcoreMesh(core_axis_name="c", subcore_axis_name="s"),
    compiler_params=pltpu.CompilerParams(use_tc_tiling_on_sc=False),
    scratch_types=[
        pltpu.VMEM((NCH * L * CH,), jnp.int32),
        pltpu.VMEM((L, CHC, CAT_DIM), jnp.float32),
        pltpu.VMEM((CH, CAT_DIM), jnp.float32),
        pltpu.SemaphoreType.DMA,
    ],
)


def kernel(movie_id, movie_categories, emb_movies_W, emb_cats_W, bias_W):
    mid = movie_id.astype(jnp.int32).reshape(NW, NCH, CH)
    mct = (movie_categories.astype(jnp.int32).T
           .reshape(L, NW, NCH, CH).transpose(1, 2, 0, 3)
           .reshape(NW, NCH * L * CH))
    mv, bias = _movie_call(mid, emb_movies_W, bias_W.T)
    (cat,) = _cat_call(mct, emb_cats_W)
    return jnp.concatenate([mv, cat], axis=1), bias
